# baseline (speedup 1.0000x reference)
import functools

import jax
import jax.numpy as jnp
from jax import lax
from jax.experimental import pallas as pl
from jax.experimental.pallas import tpu as pltpu
from jax.experimental.pallas import tpu_sc as plsc


def _sc_gating_kernel(x_hbm, wg_hbm, bg_hbm, idx_hbm, w_hbm,
                      chunk_v, idxz_v, zv_v, xm_v, wg_v, bg_v,
                      stage_i, stage_w, shared, sem, *,
                      seq, d, e, n_sub, rows_per_chunk):
    c = lax.axis_index("c")
    sid = lax.axis_index("s")
    lanes = 16

    for j in range(d // lanes):
        zv_v[0, pl.ds(j * lanes, lanes)] = jnp.zeros((lanes,), jnp.float32)
    idxz_v[...] = jnp.zeros((rows_per_chunk,), jnp.int32)

    rows_per_sub = seq // n_sub
    n_chunks = rows_per_sub // rows_per_chunk
    base = sid * rows_per_sub
    def chunk_body(j):
        r0 = base + j * rows_per_chunk
        pltpu.async_copy(
            x_hbm.at[c, pl.ds(r0, rows_per_chunk)], chunk_v, sem).wait()

        def row_body(r):
            for col in range(d // lanes):
                sl = pl.ds(col * lanes, lanes)
                zv_v[0, sl] = zv_v[0, sl] + chunk_v[r, sl]
        pl.loop(0, rows_per_chunk)(row_body)
    pl.loop(0, n_chunks)(chunk_body)
    pltpu.sync_copy(zv_v, shared.at[pl.ds(sid, 1)])
    plsc.subcore_barrier()

    @pl.when(sid == 0)
    def _():
        pltpu.sync_copy(shared, xm_v)
        pltpu.sync_copy(wg_hbm, wg_v)
        pltpu.sync_copy(bg_hbm, bg_v)
        inv_seq = 1.0 / seq
        bg_all = bg_v[...]
        def comb_body(col):
            sl = pl.ds(col * lanes, lanes)
            xsum = xm_v[0, sl]
            for r in range(1, n_sub):
                xsum = xsum + xm_v[r, sl]
            xm_v[0, sl] = xsum
        pl.loop(0, d // lanes)(comb_body)
        logits = []
        for ei in range(e):
            acc = jnp.zeros((lanes,), jnp.float32)
            for col in range(d // lanes):
                sl = pl.ds(col * lanes, lanes)
                acc = acc + xm_v[0, sl] * wg_v[ei, sl]
            total = acc[0]
            for ln in range(1, lanes):
                total = total + acc[ln]
            logits.append(total * inv_seq + bg_all[ei])
        m1 = logits[0]
        i1 = jnp.int32(0)
        for ei in range(1, e):
            better = logits[ei] > m1
            m1 = jnp.where(better, logits[ei], m1)
            i1 = jnp.where(better, jnp.int32(ei), i1)
        m2 = jnp.float32(-jnp.inf)
        i2 = jnp.int32(0)
        for ei in range(e):
            better = jnp.logical_and(i1 != ei, logits[ei] > m2)
            m2 = jnp.where(better, logits[ei], m2)
            i2 = jnp.where(better, jnp.int32(ei), i2)
        e2 = jnp.exp(jnp.zeros((lanes,), jnp.float32) + (m2 - m1))
        ga = 1.0 / (1.0 + e2)
        gb = 1.0 - ga
        ids = lax.iota(jnp.int32, lanes)
        i1v = jnp.zeros((lanes,), jnp.int32) + i1
        i2v = jnp.zeros((lanes,), jnp.int32) + i2
        stage_i[...] = jnp.where(ids == 0, i1v,
                                 jnp.where(ids == 1, i2v,
                                           jnp.zeros((lanes,), jnp.int32)))
        stage_w[...] = jnp.where(ids == 0, ga,
                                 jnp.where(ids == 1, gb,
                                           jnp.zeros((lanes,), jnp.float32)))
        pltpu.sync_copy(stage_i, idx_hbm.at[c])
        pltpu.sync_copy(stage_w, w_hbm.at[c])


def _sc_gating(x, Wg, bg):
    b_sz, seq, d = x.shape
    e = Wg.shape[0]
    n_sub = 16
    rows_per_chunk = 16
    mesh = plsc.VectorSubcoreMesh(core_axis_name="c", subcore_axis_name="s")
    bg16 = jnp.pad(bg, (0, 16 - e)) if e < 16 else bg
    idx16, w16 = pl.kernel(
        functools.partial(_sc_gating_kernel, seq=seq, d=d, e=e, n_sub=n_sub,
                          rows_per_chunk=rows_per_chunk),
        out_type=[
            jax.ShapeDtypeStruct((b_sz, 16), jnp.int32),
            jax.ShapeDtypeStruct((b_sz, 16), jnp.float32),
        ],
        mesh=mesh,
        scratch_types=[
            pltpu.VMEM((rows_per_chunk, d), jnp.float32),
            pltpu.VMEM((rows_per_chunk,), jnp.int32),
            pltpu.VMEM((1, d), jnp.float32),
            pltpu.VMEM((n_sub, d), jnp.float32),
            pltpu.VMEM((e, d), jnp.float32),
            pltpu.VMEM((16,), jnp.float32),
            pltpu.VMEM((16,), jnp.int32),
            pltpu.VMEM((16,), jnp.float32),
            pltpu.VMEM_SHARED((n_sub, d), jnp.float32),
            pltpu.SemaphoreType.DMA,
        ],
    )(x, Wg, bg16)
    return idx16[:, :2], w16[:, :2]


def _gelu_exact(v):
    return 0.5 * v * (1.0 + jax.lax.erf(v * 0.7071067811865476))


def _expert_kernel(idx_ref, w_ref, x_ref, w0_ref, w1_ref, b0_ref, b1_ref,
                   out_ref):
    b = pl.program_id(0)
    xb = x_ref[0]
    dn = (((1,), (1,)), ((), ()))
    y0 = jax.lax.dot_general(xb, w0_ref[0], dn,
                             preferred_element_type=jnp.float32)
    y0 = _gelu_exact(y0 + b0_ref[0])
    y1 = jax.lax.dot_general(xb, w1_ref[0], dn,
                             preferred_element_type=jnp.float32)
    y1 = _gelu_exact(y1 + b1_ref[0])
    out_ref[0] = w_ref[b, 0] * y0 + w_ref[b, 1] * y1


def kernel(x, Wg, bg, Wexp, bexp):
    b_sz, seq, d = x.shape
    e, o, _ = Wexp.shape

    idx, w = _sc_gating(x, Wg, bg)

    bs = 1024
    n_s = seq // bs
    grid_spec = pltpu.PrefetchScalarGridSpec(
        num_scalar_prefetch=2,
        grid=(b_sz, n_s),
        in_specs=[
            pl.BlockSpec((1, bs, d), lambda b, s, idx, w: (b, s, 0)),
            pl.BlockSpec((1, o, d), lambda b, s, idx, w: (idx[b, 0], 0, 0)),
            pl.BlockSpec((1, o, d), lambda b, s, idx, w: (idx[b, 1], 0, 0)),
            pl.BlockSpec((1, 1, o), lambda b, s, idx, w: (idx[b, 0], 0, 0)),
            pl.BlockSpec((1, 1, o), lambda b, s, idx, w: (idx[b, 1], 0, 0)),
        ],
        out_specs=pl.BlockSpec((1, bs, o), lambda b, s, idx, w: (b, s, 0)),
    )
    out = pl.pallas_call(
        _expert_kernel,
        grid_spec=grid_spec,
        out_shape=jax.ShapeDtypeStruct((b_sz, seq, o), jnp.float32),
    )(idx, w, x, Wexp, Wexp, bexp.reshape(e, 1, o), bexp.reshape(e, 1, o))
    return out


# speedup vs baseline: 2.7918x; 2.7918x over previous
import functools

import jax
import jax.numpy as jnp
from jax.experimental import pallas as pl
from jax.experimental.pallas import tpu as pltpu

def _gating_kernel(x_ref, wg_ref, bg_ref, idx_ref, w_ref, xsum_ref, *, n_s, seq):
    s = pl.program_id(0)

    @pl.when(s == 0)
    def _():
        xsum_ref[...] = jnp.zeros_like(xsum_ref)

    xsum_ref[...] += jnp.sum(x_ref[...], axis=1)

    @pl.when(s == n_s - 1)
    def _():
        xm = xsum_ref[...] / seq
        logits = jax.lax.dot_general(
            xm, wg_ref[...], (((1,), (1,)), ((), ())),
            preferred_element_type=jnp.float32) + bg_ref[...]
        e = logits.shape[1]
        ids = jax.lax.broadcasted_iota(jnp.int32, logits.shape, 1)
        m1 = jnp.max(logits, axis=1, keepdims=True)
        i1 = jnp.min(jnp.where(logits == m1, ids, e), axis=1, keepdims=True)
        rest = jnp.where(ids == i1, -jnp.inf, logits)
        m2 = jnp.max(rest, axis=1, keepdims=True)
        i2 = jnp.min(jnp.where(rest == m2, ids, e), axis=1, keepdims=True)
        idx_ref[...] = jnp.concatenate([i1, i2], axis=1)
        e2 = jnp.exp(m2 - m1)
        denom = 1.0 + e2
        w_ref[...] = jnp.concatenate([1.0 / denom, e2 / denom], axis=1)


def _gelu_exact(v):
    return 0.5 * v * (1.0 + jax.lax.erf(v * 0.7071067811865476))


def _expert_kernel(idx_ref, w_ref, x_ref, w0_ref, w1_ref, b0_ref, b1_ref,
                   out_ref):
    b = pl.program_id(0)
    xb = x_ref[0]
    dn = (((1,), (1,)), ((), ()))
    y0 = jax.lax.dot_general(xb, w0_ref[0], dn,
                             preferred_element_type=jnp.float32)
    y0 = _gelu_exact(y0 + b0_ref[0])
    y1 = jax.lax.dot_general(xb, w1_ref[0], dn,
                             preferred_element_type=jnp.float32)
    y1 = _gelu_exact(y1 + b1_ref[0])
    out_ref[0] = w_ref[b, 0] * y0 + w_ref[b, 1] * y1


def kernel(x, Wg, bg, Wexp, bexp):
    b_sz, seq, d = x.shape
    e, o, _ = Wexp.shape
    k = 2

    bs_g = 512
    n_sg = seq // bs_g
    idx, w = pl.pallas_call(
        functools.partial(_gating_kernel, n_s=n_sg, seq=seq),
        grid=(n_sg,),
        in_specs=[
            pl.BlockSpec((b_sz, bs_g, d), lambda s: (0, s, 0)),
            pl.BlockSpec((e, d), lambda s: (0, 0)),
            pl.BlockSpec((1, e), lambda s: (0, 0)),
        ],
        out_specs=[
            pl.BlockSpec((b_sz, k), lambda s: (0, 0)),
            pl.BlockSpec((b_sz, k), lambda s: (0, 0)),
        ],
        out_shape=[
            jax.ShapeDtypeStruct((b_sz, k), jnp.int32),
            jax.ShapeDtypeStruct((b_sz, k), jnp.float32),
        ],
        scratch_shapes=[pltpu.VMEM((b_sz, d), jnp.float32)],
    )(x, Wg, bg.reshape(1, e))

    bs = 1024
    n_s = seq // bs
    grid_spec = pltpu.PrefetchScalarGridSpec(
        num_scalar_prefetch=2,
        grid=(b_sz, n_s),
        in_specs=[
            pl.BlockSpec((1, bs, d), lambda b, s, idx, w: (b, s, 0)),
            pl.BlockSpec((1, o, d), lambda b, s, idx, w: (idx[b, 0], 0, 0)),
            pl.BlockSpec((1, o, d), lambda b, s, idx, w: (idx[b, 1], 0, 0)),
            pl.BlockSpec((1, 1, o), lambda b, s, idx, w: (idx[b, 0], 0, 0)),
            pl.BlockSpec((1, 1, o), lambda b, s, idx, w: (idx[b, 1], 0, 0)),
        ],
        out_specs=pl.BlockSpec((1, bs, o), lambda b, s, idx, w: (b, s, 0)),
    )
    out = pl.pallas_call(
        _expert_kernel,
        grid_spec=grid_spec,
        out_shape=jax.ShapeDtypeStruct((b_sz, seq, o), jnp.float32),
    )(idx, w, x, Wexp, Wexp, bexp.reshape(e, 1, o), bexp.reshape(e, 1, o))
    return out


# speedup vs baseline: 2.7994x; 1.0027x over previous
import functools

import jax
import jax.numpy as jnp
from jax.experimental import pallas as pl
from jax.experimental.pallas import tpu as pltpu

def _gating_kernel(x_ref, wg_ref, bg_ref, idx_ref, w_ref, xsum_ref, *, n_s, seq):
    s = pl.program_id(0)

    @pl.when(s == 0)
    def _():
        xsum_ref[...] = jnp.zeros_like(xsum_ref)

    xsum_ref[...] += jnp.sum(x_ref[...], axis=1)

    @pl.when(s == n_s - 1)
    def _():
        xm = xsum_ref[...] / seq
        logits = jax.lax.dot_general(
            xm, wg_ref[...], (((1,), (1,)), ((), ())),
            preferred_element_type=jnp.float32) + bg_ref[...]
        e = logits.shape[1]
        ids = jax.lax.broadcasted_iota(jnp.int32, logits.shape, 1)
        m1 = jnp.max(logits, axis=1, keepdims=True)
        i1 = jnp.min(jnp.where(logits == m1, ids, e), axis=1, keepdims=True)
        rest = jnp.where(ids == i1, -jnp.inf, logits)
        m2 = jnp.max(rest, axis=1, keepdims=True)
        i2 = jnp.min(jnp.where(rest == m2, ids, e), axis=1, keepdims=True)
        idx_ref[...] = jnp.concatenate([i1, i2], axis=1)
        e2 = jnp.exp(m2 - m1)
        denom = 1.0 + e2
        w_ref[...] = jnp.concatenate([1.0 / denom, e2 / denom], axis=1)


def _gelu_exact(v):
    return 0.5 * v * (1.0 + jax.lax.erf(v * 0.7071067811865476))


def _expert_kernel(idx_ref, w_ref, x_ref, w0_ref, w1_ref, b0_ref, b1_ref,
                   out_ref):
    b = pl.program_id(0)
    xb = x_ref[0]
    dn = (((1,), (1,)), ((), ()))
    y0 = jax.lax.dot_general(xb, w0_ref[0], dn,
                             preferred_element_type=jnp.float32)
    y0 = _gelu_exact(y0 + b0_ref[0])
    y1 = jax.lax.dot_general(xb, w1_ref[0], dn,
                             preferred_element_type=jnp.float32)
    y1 = _gelu_exact(y1 + b1_ref[0])
    out_ref[0] = w_ref[b, 0] * y0 + w_ref[b, 1] * y1


def kernel(x, Wg, bg, Wexp, bexp):
    b_sz, seq, d = x.shape
    e, o, _ = Wexp.shape
    k = 2

    bs_g = 1024
    n_sg = seq // bs_g
    idx, w = pl.pallas_call(
        functools.partial(_gating_kernel, n_s=n_sg, seq=seq),
        grid=(n_sg,),
        in_specs=[
            pl.BlockSpec((b_sz, bs_g, d), lambda s: (0, s, 0)),
            pl.BlockSpec((e, d), lambda s: (0, 0)),
            pl.BlockSpec((1, e), lambda s: (0, 0)),
        ],
        out_specs=[
            pl.BlockSpec((b_sz, k), lambda s: (0, 0)),
            pl.BlockSpec((b_sz, k), lambda s: (0, 0)),
        ],
        out_shape=[
            jax.ShapeDtypeStruct((b_sz, k), jnp.int32),
            jax.ShapeDtypeStruct((b_sz, k), jnp.float32),
        ],
        scratch_shapes=[pltpu.VMEM((b_sz, d), jnp.float32)],
    )(x, Wg, bg.reshape(1, e))

    bs = 1024
    n_s = seq // bs
    grid_spec = pltpu.PrefetchScalarGridSpec(
        num_scalar_prefetch=2,
        grid=(b_sz, n_s),
        in_specs=[
            pl.BlockSpec((1, bs, d), lambda b, s, idx, w: (b, s, 0)),
            pl.BlockSpec((1, o, d), lambda b, s, idx, w: (idx[b, 0], 0, 0)),
            pl.BlockSpec((1, o, d), lambda b, s, idx, w: (idx[b, 1], 0, 0)),
            pl.BlockSpec((1, 1, o), lambda b, s, idx, w: (idx[b, 0], 0, 0)),
            pl.BlockSpec((1, 1, o), lambda b, s, idx, w: (idx[b, 1], 0, 0)),
        ],
        out_specs=pl.BlockSpec((1, bs, o), lambda b, s, idx, w: (b, s, 0)),
    )
    out = pl.pallas_call(
        _expert_kernel,
        grid_spec=grid_spec,
        out_shape=jax.ShapeDtypeStruct((b_sz, seq, o), jnp.float32),
    )(idx, w, x, Wexp, Wexp, bexp.reshape(e, 1, o), bexp.reshape(e, 1, o))
    return out
